# baseline (speedup 1.0000x reference)
import functools

import jax
import jax.numpy as jnp
from jax import lax
from jax.experimental import pallas as pl
from jax.experimental.pallas import tpu as pltpu
from jax.experimental.pallas import tpu_sc as plsc

_B, _P, _C = 64, 8732, 21
_NW = 32
_RW = _P * _C
_WP = (_B // _NW) * _P
_NP = 768
_NG = _NP // 16
_CW = _NP * _C
_NFULL = _WP // _NP
_TAILP = _WP - _NFULL * _NP
_TAILW = _TAILP * _C
_TG = _TAILP // 16
_EPS = 1e-7
_LN2 = 0.6931471805599453


def _sclog(x):
    bits = plsc.bitcast(x, jnp.int32)
    e = jnp.right_shift(bits, 23) - 127
    m = plsc.bitcast(
        jnp.bitwise_or(jnp.bitwise_and(bits, 0x7FFFFF), 0x3F800000),
        jnp.float32)
    big = m > 1.4142135
    m = jnp.where(big, m * 0.5, m)
    e = jnp.where(big, e + 1, e)
    w = (m - 1.0) / (m + 1.0)
    w2 = w * w
    p = w2 * (2.0 / 7.0) + (2.0 / 5.0)
    p = p * w2 + (2.0 / 3.0)
    p = p * w2 + 2.0
    return p * w + e.astype(jnp.float32) * _LN2


def _group_masks(xb, sb, widx, valid):
    x0 = plsc.load_gather(xb, [widx], mask=valid)
    s0 = plsc.load_gather(sb, [widx], mask=valid)
    mx = x0
    ms = s0
    for c in range(1, _C):
        mx = jnp.maximum(mx, plsc.load_gather(xb, [widx + c], mask=valid))
        ms = jnp.maximum(ms, plsc.load_gather(sb, [widx + c], mask=valid))
    lm = mx > x0
    rm = ms > s0
    ol = jnp.logical_and(jnp.logical_and(lm, jnp.logical_not(rm)), valid)
    orr = jnp.logical_and(jnp.logical_and(rm, jnp.logical_not(lm)), valid)
    return ol, orr


def _compact(idx_ref, off, pidx, msk):
    mi = msk.astype(jnp.int32)
    cs = plsc.cumsum(mi)
    plsc.store_scatter(idx_ref, [off + cs - mi], pidx, mask=msk)
    return off + jnp.sum(mi)


def _masked_kl(idx_ref, n, tgt_ref, ib, acc):
    lane = lax.iota(jnp.int32, 16)

    def body(j, a):
        pos = lane + j * 16
        valid = pos < n
        rid = plsc.load_gather(idx_ref, [pos], mask=valid)
        bi = rid * _C
        g = jnp.zeros((16,), jnp.float32)
        for c in range(_C):
            t = plsc.load_gather(tgt_ref, [bi + c], mask=valid) + _EPS
            iv = plsc.load_gather(ib, [bi + c], mask=valid) + _EPS
            g = g + t * _sclog(t / iv)
        return a + jnp.where(valid, g, 0.0)

    nj = (n + 15) // 16
    return lax.fori_loop(0, nj, body, acc)


def _chunk_compute(xb, sb, ib, idx_l, idx_r, carry, ngroups, tail8):
    acc_l, acc_r, cnt_l, cnt_r = carry
    lane = lax.iota(jnp.int32, 16)
    true16 = lane < 16

    def g_body(g, offs):
        off_l, off_r = offs
        widx = lane * _C + g * (16 * _C)
        ol, orr = _group_masks(xb, sb, widx, true16)
        pidx = lane + g * 16
        return _compact(idx_l, off_l, pidx, ol), _compact(idx_r, off_r, pidx, orr)

    n_l, n_r = lax.fori_loop(0, ngroups, g_body, (jnp.int32(0), jnp.int32(0)))

    if tail8:
        base = _TAILP - 16
        widx = lane * _C + base * _C
        vmask = lane >= 8
        ol, orr = _group_masks(xb, sb, widx, vmask)
        pidx = lane + base
        n_l = _compact(idx_l, n_l, pidx, ol)
        n_r = _compact(idx_r, n_r, pidx, orr)

    acc_l = _masked_kl(idx_l, n_l, xb, ib, acc_l)
    acc_r = _masked_kl(idx_r, n_r, sb, ib, acc_r)
    return acc_l, acc_r, cnt_l + n_l, cnt_r + n_r


def _sc_body(conf_hbm, shuf_hbm, interp_hbm, out_hbm,
             xb0, sb0, ib0, xb1, sb1, ib1, idx_l, idx_r, outv,
             sx0, ss0, si0, sx1, ss1, si1):
    cid = lax.axis_index("c")
    sid = lax.axis_index("s")
    wid = sid * 2 + cid
    xbase = wid * (2 * _RW)
    sbase = ((2 * wid + 32) % _B) * _RW

    def start(k, bufs, sems, nwords):
        xb, sb, ib = bufs
        sx, ss, si = sems
        off = k * _CW
        pltpu.make_async_copy(
            conf_hbm.at[pl.ds(xbase + off, nwords)], xb.at[pl.ds(0, nwords)], sx).start()
        pltpu.make_async_copy(
            shuf_hbm.at[pl.ds(sbase + off, nwords)], sb.at[pl.ds(0, nwords)], ss).start()
        pltpu.make_async_copy(
            interp_hbm.at[pl.ds(xbase + off, nwords)], ib.at[pl.ds(0, nwords)], si).start()

    def wait(k, bufs, sems, nwords):
        xb, sb, ib = bufs
        sx, ss, si = sems
        off = k * _CW
        pltpu.make_async_copy(
            conf_hbm.at[pl.ds(xbase + off, nwords)], xb.at[pl.ds(0, nwords)], sx).wait()
        pltpu.make_async_copy(
            shuf_hbm.at[pl.ds(sbase + off, nwords)], sb.at[pl.ds(0, nwords)], ss).wait()
        pltpu.make_async_copy(
            interp_hbm.at[pl.ds(xbase + off, nwords)], ib.at[pl.ds(0, nwords)], si).wait()

    bufs0 = (xb0, sb0, ib0)
    bufs1 = (xb1, sb1, ib1)
    sems0 = (sx0, ss0, si0)
    sems1 = (sx1, ss1, si1)

    start(0, bufs0, sems0, _CW)
    start(1, bufs1, sems1, _CW)

    zero = jnp.zeros((16,), jnp.float32)
    carry0 = (zero, zero, jnp.int32(0), jnp.int32(0))

    def outer(k, carry):
        wait(2 * k, bufs0, sems0, _CW)
        carry = _chunk_compute(xb0, sb0, ib0, idx_l, idx_r, carry, _NG, False)

        @pl.when(k < _NFULL // 2 - 1)
        def _():
            start(2 * k + 2, bufs0, sems0, _CW)

        @pl.when(k == _NFULL // 2 - 1)
        def _():
            start(_NFULL, bufs0, sems0, _TAILW)

        wait(2 * k + 1, bufs1, sems1, _CW)
        carry = _chunk_compute(xb1, sb1, ib1, idx_l, idx_r, carry, _NG, False)

        @pl.when(k < _NFULL // 2 - 1)
        def _():
            start(2 * k + 3, bufs1, sems1, _CW)

        return carry

    carry = lax.fori_loop(0, _NFULL // 2, outer, carry0)

    wait(_NFULL, bufs0, sems0, _TAILW)
    acc_l, acc_r, cnt_l, cnt_r = _chunk_compute(
        xb0, sb0, ib0, idx_l, idx_r, carry, _TG, True)

    lane = lax.iota(jnp.int32, 16)
    sum_l = jnp.sum(acc_l)
    sum_r = jnp.sum(acc_r)
    row = jnp.where(lane == 0, sum_l, 0.0)
    row = jnp.where(lane == 1, cnt_l.astype(jnp.float32), row)
    row = jnp.where(lane == 2, sum_r, row)
    row = jnp.where(lane == 3, cnt_r.astype(jnp.float32), row)
    outv[...] = row
    pltpu.sync_copy(outv, out_hbm.at[wid])


def _fin_body(p_ref, o_ref):
    r = p_ref[...]
    sl = jnp.sum(r[:, 0])
    cl = jnp.sum(r[:, 1])
    sr = jnp.sum(r[:, 2])
    cr = jnp.sum(r[:, 3])
    ll = jnp.where(cl > 0.0, sl / jnp.maximum(cl, 1.0), 0.0)
    lr = jnp.where(cr > 0.0, sr / jnp.maximum(cr, 1.0), 0.0)
    o_ref[0] = ll + lr


@functools.partial(
    pl.kernel,
    out_type=jax.ShapeDtypeStruct((_NW, 16), jnp.float32),
    mesh=plsc.VectorSubcoreMesh(core_axis_name="c", subcore_axis_name="s",
                                num_cores=2, num_subcores=16),
    compiler_params=pltpu.CompilerParams(needs_layout_passes=False),
    scratch_types=[
        pltpu.VMEM((_CW,), jnp.float32),
        pltpu.VMEM((_CW,), jnp.float32),
        pltpu.VMEM((_CW,), jnp.float32),
        pltpu.VMEM((_CW,), jnp.float32),
        pltpu.VMEM((_CW,), jnp.float32),
        pltpu.VMEM((_CW,), jnp.float32),
        pltpu.VMEM((_NP,), jnp.int32),
        pltpu.VMEM((_NP,), jnp.int32),
        pltpu.VMEM((16,), jnp.float32),
        pltpu.SemaphoreType.DMA,
        pltpu.SemaphoreType.DMA,
        pltpu.SemaphoreType.DMA,
        pltpu.SemaphoreType.DMA,
        pltpu.SemaphoreType.DMA,
        pltpu.SemaphoreType.DMA,
    ],
)
def _sc_kernel(conf_hbm, shuf_hbm, interp_hbm, out_hbm, *rest):
    _sc_body(conf_hbm, shuf_hbm, interp_hbm, out_hbm, *rest)


def kernel(args, lam, conf, conf_flip, loc, loc_flip, conf_shuffle,
           conf_interpolation, loc_shuffle, loc_interpolation):
    partials = _sc_kernel(
        conf.reshape(-1), conf_shuffle.reshape(-1),
        conf_interpolation.reshape(-1))
    loss = pl.pallas_call(
        _fin_body,
        out_specs=pl.BlockSpec(memory_space=pltpu.SMEM),
        out_shape=jax.ShapeDtypeStruct((1,), jnp.float32),
    )(partials)
    return (jnp.zeros((1,), jnp.float32), loss[0])


# speedup vs baseline: 3.3720x; 3.3720x over previous
import functools

import jax
import jax.numpy as jnp
from jax import lax
from jax.experimental import pallas as pl
from jax.experimental.pallas import tpu as pltpu
from jax.experimental.pallas import tpu_sc as plsc

_B, _P, _C = 64, 8732, 21
_NW = 32
_NP = 128
_NG = _NP // 16
_KF = _P // _NP
_TAILP = _P - _KF * _NP
_TG = _TAILP // 16
_TREM = _TAILP - _TG * 16
_NFULL = 2 * _KF
_EPS = 1e-7
_LN2 = 0.6931471805599453


def _sclog(x):
    bits = plsc.bitcast(x, jnp.int32)
    e = jnp.right_shift(bits, 23) - 127
    m = plsc.bitcast(
        jnp.bitwise_or(jnp.bitwise_and(bits, 0x7FFFFF), 0x3F800000),
        jnp.float32)
    big = m > 1.4142135
    m = jnp.where(big, m * 0.5, m)
    e = jnp.where(big, e + 1, e)
    w = (m - 1.0) / (m + 1.0)
    w2 = w * w
    p = w2 * (2.0 / 7.0) + (2.0 / 5.0)
    p = p * w2 + (2.0 / 3.0)
    p = p * w2 + 2.0
    return p * w + e.astype(jnp.float32) * _LN2


def _group_masks(xb, sb, pidx, valid):
    c0 = jnp.zeros((16,), jnp.int32)
    x0 = plsc.load_gather(xb, [pidx, c0], mask=valid)
    s0 = plsc.load_gather(sb, [pidx, c0], mask=valid)
    mx = x0
    ms = s0
    for c in range(1, _C):
        cc = jnp.full((16,), c, jnp.int32)
        mx = jnp.maximum(mx, plsc.load_gather(xb, [pidx, cc], mask=valid))
        ms = jnp.maximum(ms, plsc.load_gather(sb, [pidx, cc], mask=valid))
    lm = mx > x0
    rm = ms > s0
    ol = jnp.logical_and(jnp.logical_and(lm, jnp.logical_not(rm)), valid)
    orr = jnp.logical_and(jnp.logical_and(rm, jnp.logical_not(lm)), valid)
    return ol, orr


def _compact(idx_ref, off, pidx, msk):
    mi = msk.astype(jnp.int32)
    cs = plsc.cumsum(mi)
    plsc.store_scatter(idx_ref, [off + cs - mi], pidx, mask=msk)
    return off + jnp.sum(mi)


def _masked_kl(idx_ref, n, tgt_ref, ib, acc):
    lane = lax.iota(jnp.int32, 16)

    def body(j, a):
        pos = lane + j * 16
        valid = pos < n
        rid = plsc.load_gather(idx_ref, [pos], mask=valid)
        g = jnp.zeros((16,), jnp.float32)
        for c in range(_C):
            cc = jnp.full((16,), c, jnp.int32)
            t = plsc.load_gather(tgt_ref, [rid, cc], mask=valid) + _EPS
            iv = plsc.load_gather(ib, [rid, cc], mask=valid) + _EPS
            g = g + t * _sclog(t / iv)
        return a + jnp.where(valid, g, 0.0)

    nj = (n + 15) // 16
    return lax.fori_loop(0, nj, body, acc)


def _chunk_compute(xb, sb, ib, idx_l, idx_r, carry, ngroups, tail):
    acc_l, acc_r, cnt_l, cnt_r = carry
    lane = lax.iota(jnp.int32, 16)
    true16 = lane < 16

    def g_body(g, offs):
        off_l, off_r = offs
        pidx = lane + g * 16
        ol, orr = _group_masks(xb, sb, pidx, true16)
        return _compact(idx_l, off_l, pidx, ol), _compact(idx_r, off_r, pidx, orr)

    n_l, n_r = lax.fori_loop(0, ngroups, g_body, (jnp.int32(0), jnp.int32(0)))

    if tail:
        pidx = lane + (_TAILP - 16)
        vmask = lane >= (16 - _TREM)
        ol, orr = _group_masks(xb, sb, pidx, vmask)
        n_l = _compact(idx_l, n_l, pidx, ol)
        n_r = _compact(idx_r, n_r, pidx, orr)

    acc_l = _masked_kl(idx_l, n_l, xb, ib, acc_l)
    acc_r = _masked_kl(idx_r, n_r, sb, ib, acc_r)
    return acc_l, acc_r, cnt_l + n_l, cnt_r + n_r


def _sc_body(conf_hbm, shuf_hbm, interp_hbm, out_hbm, outv,
             sx0, ss0, si0, sx1, ss1, si1):
    pl.run_scoped(
        functools.partial(_sc_inner, conf_hbm, shuf_hbm, interp_hbm, out_hbm,
                          outv, sx0, ss0, si0, sx1, ss1, si1),
        pltpu.VMEM((_NP, _C), jnp.float32),
        pltpu.VMEM((_NP, _C), jnp.float32),
        pltpu.VMEM((_NP, _C), jnp.float32),
        pltpu.VMEM((_NP, _C), jnp.float32),
        pltpu.VMEM((_NP, _C), jnp.float32),
        pltpu.VMEM((_NP, _C), jnp.float32),
        pltpu.VMEM((_NP,), jnp.int32),
        pltpu.VMEM((_NP,), jnp.int32),
    )


def _sc_inner(conf_hbm, shuf_hbm, interp_hbm, out_hbm, outv,
              sx0, ss0, si0, sx1, ss1, si1,
              xb0, sb0, ib0, xb1, sb1, ib1, idx_l, idx_r):
    cid = lax.axis_index("c")
    sid = lax.axis_index("s")
    wid = sid * 2 + cid
    brow = 2 * wid
    srow = (2 * wid + 32) % _B

    def start(c, bufs, sems, np_):
        xb, sb, ib = bufs
        sx, ss, si = sems
        row = c // _KF
        p0 = (c % _KF) * _NP
        pltpu.make_async_copy(
            conf_hbm.at[brow + row, pl.ds(p0, np_), :],
            xb.at[pl.ds(0, np_), :], sx).start()
        pltpu.make_async_copy(
            shuf_hbm.at[srow + row, pl.ds(p0, np_), :],
            sb.at[pl.ds(0, np_), :], ss).start()
        pltpu.make_async_copy(
            interp_hbm.at[brow + row, pl.ds(p0, np_), :],
            ib.at[pl.ds(0, np_), :], si).start()

    def wait(c, bufs, sems, np_):
        xb, sb, ib = bufs
        sx, ss, si = sems
        row = c // _KF
        p0 = (c % _KF) * _NP
        pltpu.make_async_copy(
            conf_hbm.at[brow + row, pl.ds(p0, np_), :],
            xb.at[pl.ds(0, np_), :], sx).wait()
        pltpu.make_async_copy(
            shuf_hbm.at[srow + row, pl.ds(p0, np_), :],
            sb.at[pl.ds(0, np_), :], ss).wait()
        pltpu.make_async_copy(
            interp_hbm.at[brow + row, pl.ds(p0, np_), :],
            ib.at[pl.ds(0, np_), :], si).wait()

    bufs0 = (xb0, sb0, ib0)
    bufs1 = (xb1, sb1, ib1)
    sems0 = (sx0, ss0, si0)
    sems1 = (sx1, ss1, si1)

    def start_tail(row, bufs, sems):
        xb, sb, ib = bufs
        sx, ss, si = sems
        p0 = _KF * _NP
        pltpu.make_async_copy(
            conf_hbm.at[brow + row, pl.ds(p0, _TAILP), :],
            xb.at[pl.ds(0, _TAILP), :], sx).start()
        pltpu.make_async_copy(
            shuf_hbm.at[srow + row, pl.ds(p0, _TAILP), :],
            sb.at[pl.ds(0, _TAILP), :], ss).start()
        pltpu.make_async_copy(
            interp_hbm.at[brow + row, pl.ds(p0, _TAILP), :],
            ib.at[pl.ds(0, _TAILP), :], si).start()

    def wait_tail(row, bufs, sems):
        xb, sb, ib = bufs
        sx, ss, si = sems
        p0 = _KF * _NP
        pltpu.make_async_copy(
            conf_hbm.at[brow + row, pl.ds(p0, _TAILP), :],
            xb.at[pl.ds(0, _TAILP), :], sx).wait()
        pltpu.make_async_copy(
            shuf_hbm.at[srow + row, pl.ds(p0, _TAILP), :],
            sb.at[pl.ds(0, _TAILP), :], ss).wait()
        pltpu.make_async_copy(
            interp_hbm.at[brow + row, pl.ds(p0, _TAILP), :],
            ib.at[pl.ds(0, _TAILP), :], si).wait()

    start(0, bufs0, sems0, _NP)
    start(1, bufs1, sems1, _NP)

    zero = jnp.zeros((16,), jnp.float32)
    carry0 = (zero, zero, jnp.int32(0), jnp.int32(0))

    def outer(k, carry):
        wait(2 * k, bufs0, sems0, _NP)
        carry = _chunk_compute(xb0, sb0, ib0, idx_l, idx_r, carry, _NG, False)

        @pl.when(k < _NFULL // 2 - 1)
        def _():
            start(2 * k + 2, bufs0, sems0, _NP)

        @pl.when(k == _NFULL // 2 - 1)
        def _():
            start_tail(0, bufs0, sems0)

        wait(2 * k + 1, bufs1, sems1, _NP)
        carry = _chunk_compute(xb1, sb1, ib1, idx_l, idx_r, carry, _NG, False)

        @pl.when(k < _NFULL // 2 - 1)
        def _():
            start(2 * k + 3, bufs1, sems1, _NP)

        @pl.when(k == _NFULL // 2 - 1)
        def _():
            start_tail(1, bufs1, sems1)

        return carry

    carry = lax.fori_loop(0, _NFULL // 2, outer, carry0)

    wait_tail(0, bufs0, sems0)
    carry = _chunk_compute(xb0, sb0, ib0, idx_l, idx_r, carry, _TG, True)
    wait_tail(1, bufs1, sems1)
    acc_l, acc_r, cnt_l, cnt_r = _chunk_compute(
        xb1, sb1, ib1, idx_l, idx_r, carry, _TG, True)

    lane = lax.iota(jnp.int32, 16)
    row = jnp.where(lane == 0, jnp.sum(acc_l), 0.0)
    row = jnp.where(lane == 1, cnt_l.astype(jnp.float32), row)
    row = jnp.where(lane == 2, jnp.sum(acc_r), row)
    row = jnp.where(lane == 3, cnt_r.astype(jnp.float32), row)
    outv[...] = row
    pltpu.sync_copy(outv, out_hbm.at[wid])


def _fin_body(p_ref, o_ref):
    r = p_ref[...]
    sl = jnp.sum(r[:, 0])
    cl = jnp.sum(r[:, 1])
    sr = jnp.sum(r[:, 2])
    cr = jnp.sum(r[:, 3])
    ll = jnp.where(cl > 0.0, sl / jnp.maximum(cl, 1.0), 0.0)
    lr = jnp.where(cr > 0.0, sr / jnp.maximum(cr, 1.0), 0.0)
    o_ref[0] = ll + lr


@functools.partial(
    pl.kernel,
    out_type=jax.ShapeDtypeStruct((_NW, 16), jnp.float32),
    mesh=plsc.VectorSubcoreMesh(core_axis_name="c", subcore_axis_name="s",
                                num_cores=2, num_subcores=16),
    compiler_params=pltpu.CompilerParams(needs_layout_passes=False),
    scratch_types=[
        pltpu.VMEM((16,), jnp.float32),
        pltpu.SemaphoreType.DMA,
        pltpu.SemaphoreType.DMA,
        pltpu.SemaphoreType.DMA,
        pltpu.SemaphoreType.DMA,
        pltpu.SemaphoreType.DMA,
        pltpu.SemaphoreType.DMA,
    ],
)
def _sc_kernel(conf_hbm, shuf_hbm, interp_hbm, out_hbm, *rest):
    _sc_body(conf_hbm, shuf_hbm, interp_hbm, out_hbm, *rest)


def kernel(args, lam, conf, conf_flip, loc, loc_flip, conf_shuffle,
           conf_interpolation, loc_shuffle, loc_interpolation):
    partials = _sc_kernel(conf, conf_shuffle, conf_interpolation)
    loss = pl.pallas_call(
        _fin_body,
        out_specs=pl.BlockSpec(memory_space=pltpu.SMEM),
        out_shape=jax.ShapeDtypeStruct((1,), jnp.float32),
    )(partials)
    return (jnp.zeros((1,), jnp.float32), loss[0])


# speedup vs baseline: 47.8783x; 14.1989x over previous
import jax
import jax.numpy as jnp
from jax.experimental import pallas as pl
from jax.experimental.pallas import tpu as pltpu

_B, _P, _C = 64, 8732, 21
_BB = 8
_NBLK = _B // _BB
_EPS = 1e-7


def _body(conf_ref, shuf_ref, interp_ref, out_ref, acc_ref):
    j = pl.program_id(0)
    first = j == 0
    last = j == _NBLK - 1

    @pl.when(first)
    def _init():
        acc_ref[0] = 0.0
        acc_ref[1] = 0.0
        acc_ref[2] = 0.0
        acc_ref[3] = 0.0

    x = conf_ref[...]
    s = shuf_ref[...]
    i = interp_ref[...]

    x0 = x[0]
    s0 = s[0]
    lm = jnp.max(x, axis=0) > x0
    rm = jnp.max(s, axis=0) > s0
    olf = jnp.logical_and(lm, jnp.logical_not(rm)).astype(jnp.float32)
    orf = jnp.logical_and(rm, jnp.logical_not(lm)).astype(jnp.float32)

    ri = 1.0 / (i + _EPS)
    tx = x + _EPS
    ts = s + _EPS
    accx = jnp.sum(tx * jnp.log(tx * ri), axis=0)
    accs = jnp.sum(ts * jnp.log(ts * ri), axis=0)

    acc_ref[0] += jnp.sum(accx * olf)
    acc_ref[1] += jnp.sum(olf)
    acc_ref[2] += jnp.sum(accs * orf)
    acc_ref[3] += jnp.sum(orf)

    @pl.when(last)
    def _fin():
        sl, cl, sr, cr = acc_ref[0], acc_ref[1], acc_ref[2], acc_ref[3]
        loss_l = jnp.where(cl > 0.0, sl / jnp.maximum(cl, 1.0), 0.0)
        loss_r = jnp.where(cr > 0.0, sr / jnp.maximum(cr, 1.0), 0.0)
        out_ref[0] = loss_l + loss_r


def kernel(args, lam, conf, conf_flip, loc, loc_flip, conf_shuffle,
           conf_interpolation, loc_shuffle, loc_interpolation):
    half_blk = (_B // 2) // _BB
    nblk = _NBLK
    conf_t = jnp.transpose(conf, (2, 0, 1))
    shuf_t = jnp.transpose(conf_shuffle, (2, 0, 1))
    interp_t = jnp.transpose(conf_interpolation, (2, 0, 1))
    loss = pl.pallas_call(
        _body,
        grid=(nblk,),
        in_specs=[
            pl.BlockSpec((_C, _BB, _P), lambda j: (0, j, 0)),
            pl.BlockSpec((_C, _BB, _P),
                         lambda j: (0, (j + half_blk) % nblk, 0)),
            pl.BlockSpec((_C, _BB, _P), lambda j: (0, j, 0)),
        ],
        out_specs=pl.BlockSpec(memory_space=pltpu.SMEM),
        out_shape=jax.ShapeDtypeStruct((1,), jnp.float32),
        scratch_shapes=[pltpu.SMEM((4,), jnp.float32)],
    )(conf_t, shuf_t, interp_t)
    return (jnp.zeros((1,), jnp.float32), loss[0])


# speedup vs baseline: 53.7366x; 1.1224x over previous
import jax
import jax.numpy as jnp
from jax.experimental import pallas as pl
from jax.experimental.pallas import tpu as pltpu

_B, _P, _C = 64, 8732, 21
_BB = 8
_NBLK = _B // _BB
_EPS = 1e-7


def _body(conf_ref, shuf_ref, interp_ref, out_ref, acc_ref):
    j = pl.program_id(0)
    first = j == 0
    last = j == _NBLK - 1

    @pl.when(first)
    def _init():
        acc_ref[0] = 0.0
        acc_ref[1] = 0.0
        acc_ref[2] = 0.0
        acc_ref[3] = 0.0

    x0 = conf_ref[0]
    s0 = shuf_ref[0]
    mx = x0
    ms = s0
    for c in range(1, _C):
        mx = jnp.maximum(mx, conf_ref[c])
        ms = jnp.maximum(ms, shuf_ref[c])
    lm = mx > x0
    rm = ms > s0
    olf = jnp.logical_and(lm, jnp.logical_not(rm)).astype(jnp.float32)
    orf = jnp.logical_and(rm, jnp.logical_not(lm)).astype(jnp.float32)

    accx = jnp.zeros_like(x0)
    accs = jnp.zeros_like(x0)
    for c in range(_C):
        ric = 1.0 / (interp_ref[c] + _EPS)
        tx = conf_ref[c] + _EPS
        ts = shuf_ref[c] + _EPS
        accx = accx + tx * jnp.log(tx * ric)
        accs = accs + ts * jnp.log(ts * ric)

    acc_ref[0] += jnp.sum(accx * olf)
    acc_ref[1] += jnp.sum(olf)
    acc_ref[2] += jnp.sum(accs * orf)
    acc_ref[3] += jnp.sum(orf)

    @pl.when(last)
    def _fin():
        sl, cl, sr, cr = acc_ref[0], acc_ref[1], acc_ref[2], acc_ref[3]
        loss_l = jnp.where(cl > 0.0, sl / jnp.maximum(cl, 1.0), 0.0)
        loss_r = jnp.where(cr > 0.0, sr / jnp.maximum(cr, 1.0), 0.0)
        out_ref[0] = loss_l + loss_r


def kernel(args, lam, conf, conf_flip, loc, loc_flip, conf_shuffle,
           conf_interpolation, loc_shuffle, loc_interpolation):
    half_blk = (_B // 2) // _BB
    nblk = _NBLK
    conf_t = jnp.transpose(conf, (2, 0, 1))
    shuf_t = jnp.transpose(conf_shuffle, (2, 0, 1))
    interp_t = jnp.transpose(conf_interpolation, (2, 0, 1))
    loss = pl.pallas_call(
        _body,
        grid=(nblk,),
        in_specs=[
            pl.BlockSpec((_C, _BB, _P), lambda j: (0, j, 0)),
            pl.BlockSpec((_C, _BB, _P),
                         lambda j: (0, (j + half_blk) % nblk, 0)),
            pl.BlockSpec((_C, _BB, _P), lambda j: (0, j, 0)),
        ],
        out_specs=pl.BlockSpec(memory_space=pltpu.SMEM),
        out_shape=jax.ShapeDtypeStruct((1,), jnp.float32),
        scratch_shapes=[pltpu.SMEM((4,), jnp.float32)],
    )(conf_t, shuf_t, interp_t)
    return (jnp.zeros((1,), jnp.float32), loss[0])


# speedup vs baseline: 54.1806x; 1.0083x over previous
import jax
import jax.numpy as jnp
from jax.experimental import pallas as pl
from jax.experimental.pallas import tpu as pltpu

_B, _P, _C = 64, 8732, 21
_BB = 8
_NBLK = _B // _BB
_EPS = 1e-7


def _body(conf_ref, shuf_ref, interp_ref, out_ref, acc_ref):
    j = pl.program_id(0)
    first = j == 0
    last = j == _NBLK - 1

    @pl.when(first)
    def _init():
        acc_ref[0] = 0.0
        acc_ref[1] = 0.0
        acc_ref[2] = 0.0
        acc_ref[3] = 0.0

    x0 = conf_ref[0]
    s0 = shuf_ref[0]
    mx = x0
    ms = s0
    accx = jnp.zeros_like(x0)
    accs = jnp.zeros_like(x0)
    for c in range(_C):
        xc = conf_ref[c]
        sc = shuf_ref[c]
        ric = 1.0 / (interp_ref[c] + _EPS)
        mx = jnp.maximum(mx, xc)
        ms = jnp.maximum(ms, sc)
        tx = xc + _EPS
        ts = sc + _EPS
        accx = accx + tx * jnp.log(tx * ric)
        accs = accs + ts * jnp.log(ts * ric)
    lm = mx > x0
    rm = ms > s0
    olf = jnp.logical_and(lm, jnp.logical_not(rm)).astype(jnp.float32)
    orf = jnp.logical_and(rm, jnp.logical_not(lm)).astype(jnp.float32)

    acc_ref[0] += jnp.sum(accx * olf)
    acc_ref[1] += jnp.sum(olf)
    acc_ref[2] += jnp.sum(accs * orf)
    acc_ref[3] += jnp.sum(orf)

    @pl.when(last)
    def _fin():
        sl, cl, sr, cr = acc_ref[0], acc_ref[1], acc_ref[2], acc_ref[3]
        loss_l = jnp.where(cl > 0.0, sl / jnp.maximum(cl, 1.0), 0.0)
        loss_r = jnp.where(cr > 0.0, sr / jnp.maximum(cr, 1.0), 0.0)
        out_ref[0] = loss_l + loss_r


def kernel(args, lam, conf, conf_flip, loc, loc_flip, conf_shuffle,
           conf_interpolation, loc_shuffle, loc_interpolation):
    half_blk = (_B // 2) // _BB
    nblk = _NBLK
    conf_t = jnp.transpose(conf, (2, 0, 1))
    shuf_t = jnp.transpose(conf_shuffle, (2, 0, 1))
    interp_t = jnp.transpose(conf_interpolation, (2, 0, 1))
    loss = pl.pallas_call(
        _body,
        grid=(nblk,),
        in_specs=[
            pl.BlockSpec((_C, _BB, _P), lambda j: (0, j, 0)),
            pl.BlockSpec((_C, _BB, _P),
                         lambda j: (0, (j + half_blk) % nblk, 0)),
            pl.BlockSpec((_C, _BB, _P), lambda j: (0, j, 0)),
        ],
        out_specs=pl.BlockSpec(memory_space=pltpu.SMEM),
        out_shape=jax.ShapeDtypeStruct((1,), jnp.float32),
        scratch_shapes=[pltpu.SMEM((4,), jnp.float32)],
    )(conf_t, shuf_t, interp_t)
    return (jnp.zeros((1,), jnp.float32), loss[0])


# speedup vs baseline: 54.2651x; 1.0016x over previous
import jax
import jax.numpy as jnp
from jax.experimental import pallas as pl
from jax.experimental.pallas import tpu as pltpu

_B, _P, _C = 64, 8732, 21
_BB = 8
_NBLK = _B // _BB
_EPS = 1e-7


def _body(conf_ref, shuf_ref, interp_ref, out_ref, acc_ref):
    j = pl.program_id(0)
    first = j == 0
    last = j == _NBLK - 1

    @pl.when(first)
    def _init():
        acc_ref[0] = 0.0
        acc_ref[1] = 0.0
        acc_ref[2] = 0.0
        acc_ref[3] = 0.0

    x0 = conf_ref[0]
    s0 = shuf_ref[0]
    mx = x0
    ms = s0
    ri0 = 1.0 / (interp_ref[0] + _EPS)
    t0 = x0 + _EPS
    u0 = s0 + _EPS
    accx = t0 * jnp.log(t0 * ri0)
    accs = u0 * jnp.log(u0 * ri0)
    for c in range(1, _C):
        xc = conf_ref[c]
        sc = shuf_ref[c]
        ric = 1.0 / (interp_ref[c] + _EPS)
        mx = jnp.maximum(mx, xc)
        ms = jnp.maximum(ms, sc)
        tx = xc + _EPS
        ts = sc + _EPS
        accx = accx + tx * jnp.log(tx * ric)
        accs = accs + ts * jnp.log(ts * ric)
    lm = mx > x0
    rm = ms > s0
    olf = jnp.logical_and(lm, jnp.logical_not(rm)).astype(jnp.float32)
    orf = jnp.logical_and(rm, jnp.logical_not(lm)).astype(jnp.float32)

    acc_ref[0] += jnp.sum(accx * olf)
    acc_ref[1] += jnp.sum(olf)
    acc_ref[2] += jnp.sum(accs * orf)
    acc_ref[3] += jnp.sum(orf)

    @pl.when(last)
    def _fin():
        sl, cl, sr, cr = acc_ref[0], acc_ref[1], acc_ref[2], acc_ref[3]
        loss_l = jnp.where(cl > 0.0, sl / jnp.maximum(cl, 1.0), 0.0)
        loss_r = jnp.where(cr > 0.0, sr / jnp.maximum(cr, 1.0), 0.0)
        out_ref[0] = loss_l + loss_r


def kernel(args, lam, conf, conf_flip, loc, loc_flip, conf_shuffle,
           conf_interpolation, loc_shuffle, loc_interpolation):
    half_blk = (_B // 2) // _BB
    nblk = _NBLK
    conf_t = jnp.transpose(conf, (2, 0, 1))
    shuf_t = jnp.transpose(conf_shuffle, (2, 0, 1))
    interp_t = jnp.transpose(conf_interpolation, (2, 0, 1))
    loss = pl.pallas_call(
        _body,
        grid=(nblk,),
        in_specs=[
            pl.BlockSpec((_C, _BB, _P), lambda j: (0, j, 0)),
            pl.BlockSpec((_C, _BB, _P),
                         lambda j: (0, (j + half_blk) % nblk, 0)),
            pl.BlockSpec((_C, _BB, _P), lambda j: (0, j, 0)),
        ],
        out_specs=pl.BlockSpec(memory_space=pltpu.SMEM),
        out_shape=jax.ShapeDtypeStruct((1,), jnp.float32),
        scratch_shapes=[pltpu.SMEM((4,), jnp.float32)],
    )(conf_t, shuf_t, interp_t)
    return (jnp.zeros((1,), jnp.float32), loss[0])
